# baseline (speedup 1.0000x reference)
import functools

import jax
import jax.numpy as jnp
from jax import lax
from jax.experimental import pallas as pl
from jax.experimental.pallas import tpu as pltpu
from jax.experimental.pallas import tpu_sc as plsc

_CH = 128
_NBUF = 5


@functools.lru_cache(maxsize=None)
def _make_lookup(B, V, D, nc, ns):
    NW = nc * ns
    b_per_w = B // NW
    n_chunks = b_per_w // _CH
    assert n_chunks % _NBUF == 0 and n_chunks >= _NBUF
    mesh = plsc.VectorSubcoreMesh(core_axis_name="c", subcore_axis_name="s")

    @functools.partial(
        pl.kernel,
        mesh=mesh,
        out_type=jax.ShapeDtypeStruct((B, D), jnp.float32),
        scratch_types=[
            pltpu.VMEM_SHARED((V, D), jnp.float32),
            pltpu.VMEM((b_per_w,), jnp.int32),
            pltpu.VMEM((_NBUF, _CH, D), jnp.float32),
        ]
        + [pltpu.SemaphoreType.DMA] * (2 * _NBUF),
    )
    def lookup_kernel(idx_hbm, table_hbm, out_hbm, table_sh, idx_v, rows_v,
                      *sems):
        sem_g = sems[:_NBUF]
        sem_s = sems[_NBUF:]
        wid = lax.axis_index("s") * nc + lax.axis_index("c")
        base = wid * b_per_w

        @pl.when(lax.axis_index("s") == 0)
        def _():
            pltpu.sync_copy(table_hbm, table_sh)

        pltpu.sync_copy(idx_hbm.at[pl.ds(base, b_per_w)], idx_v)
        plsc.subcore_barrier()

        def gather_desc(g, b):
            idx_sl = idx_v.at[pl.ds(g * _CH, _CH)]
            return pltpu.make_async_copy(
                table_sh.at[idx_sl], rows_v.at[b], sem_g[b])

        def scatter_desc(g, b):
            return pltpu.make_async_copy(
                rows_v.at[b],
                out_hbm.at[pl.ds(base + g * _CH, _CH)],
                sem_s[b])

        gather_desc(0, 0).start()
        gather_desc(1, 1).start()

        def body(gg, carry):
            for b in range(_NBUF):
                g = gg * _NBUF + b
                bg = (b + 2) % _NBUF

                @pl.when(g >= _NBUF - 2)
                def _():
                    scatter_desc(g + 2 - _NBUF, bg).wait()

                @pl.when(g + 2 < n_chunks)
                def _():
                    gather_desc(g + 2, bg).start()

                gather_desc(g, b).wait()
                scatter_desc(g, b).start()
            return carry

        lax.fori_loop(0, n_chunks // _NBUF, body, 0)
        for c in range(n_chunks - _NBUF + 2, n_chunks):
            scatter_desc(c, c % _NBUF).wait()

    return lookup_kernel


def kernel(elems, table):
    shape = elems.shape
    V, D = table.shape
    idx = elems.reshape(-1).astype(jnp.int32)
    B = idx.shape[0]
    info = plsc.get_sparse_core_info()
    nc, ns = info.num_cores, info.num_subcores
    group = nc * ns * _CH * _NBUF
    Bp = ((B + group - 1) // group) * group
    if Bp != B:
        idx = jnp.pad(idx, (0, Bp - B))
    out = _make_lookup(Bp, V, D, nc, ns)(idx, table)
    if Bp != B:
        out = out[:B]
    return out.reshape(*shape, D)
